# baseline (speedup 1.0000x reference)
import functools

import jax
import jax.numpy as jnp
from jax import lax
from jax.experimental import pallas as pl
from jax.experimental.pallas import tpu as pltpu
from jax.experimental.pallas import tpu_sc as plsc

_NUM_CLASSES = 150
_D = 768
_S = 1024
_BM = 3072


def _sc_gather_tail(labels, table, R):
    info = plsc.get_sparse_core_info()
    NC, NS = info.num_cores, info.num_subcores
    NW = NC * NS
    b_per_w = _S // NW
    CH = 32
    n_ch = b_per_w // CH
    S0 = R - _S

    mesh = plsc.VectorSubcoreMesh(core_axis_name="c", subcore_axis_name="s")

    @functools.partial(
        pl.kernel,
        mesh=mesh,
        out_type=jax.ShapeDtypeStruct((_S, _D), jnp.float32),
        scratch_types=[
            pltpu.VMEM((b_per_w,), jnp.int32),
            pltpu.VMEM((CH, _D), jnp.float32),
            pltpu.VMEM((CH, _D), jnp.float32),
            pltpu.SemaphoreType.DMA,
            pltpu.SemaphoreType.DMA,
            pltpu.SemaphoreType.DMA,
            pltpu.SemaphoreType.DMA,
        ],
    )
    def k(lab_hbm, tab_hbm, out_hbm, idx_v, b0, b1, is0, is1, os0, os1):
        bufs = (b0, b1)
        isems = (is0, is1)
        osems = (os0, os1)
        wid = lax.axis_index("s") * NC + lax.axis_index("c")
        base = wid * b_per_w
        pltpu.sync_copy(lab_hbm.at[pl.ds(S0 + base, b_per_w)], idx_v)
        prev_out = [None, None]
        for c in range(n_ch):
            i = c & 1
            r0 = base + c * CH
            if prev_out[i] is not None:
                prev_out[i].wait()
            e_cp = pltpu.make_async_copy(
                tab_hbm.at[idx_v.at[pl.ds(c * CH, CH)]], bufs[i], isems[i])
            e_cp.start()
            e_cp.wait()
            o_cp = pltpu.make_async_copy(
                bufs[i], out_hbm.at[pl.ds(r0, CH), :], osems[i])
            o_cp.start()
            prev_out[i] = o_cp
        prev_out[0].wait()
        if prev_out[1] is not None:
            prev_out[1].wait()

    return k(labels, table)


def _tc_concat_gather_head(x2, labels, table):
    R = x2.shape[0]
    nb = (R - _S) // _BM
    lab3 = labels[: R - _S].reshape(nb, 1, _BM)

    def body(lab_ref, x_ref, tab_ref, out_ref):
        out_ref[:, : _D] = x_ref[...]
        lab = lab_ref[0, 0, :]
        onehot = (lab[:, None] == lax.broadcasted_iota(
            jnp.int32, (_BM, _NUM_CLASSES), 1)).astype(jnp.float32)
        out_ref[:, _D:] = jnp.dot(
            onehot, tab_ref[...], preferred_element_type=jnp.float32)

    return pl.pallas_call(
        body,
        grid=(nb,),
        in_specs=[
            pl.BlockSpec((1, 1, _BM), lambda i: (i, 0, 0)),
            pl.BlockSpec((_BM, _D), lambda i: (i, 0)),
            pl.BlockSpec((_NUM_CLASSES, _D), lambda i: (0, 0)),
        ],
        out_specs=pl.BlockSpec((_BM, 2 * _D), lambda i: (i, 0)),
        out_shape=jax.ShapeDtypeStruct((R, 2 * _D), jnp.float32),
    )(lab3, x2, table)


def _tc_merge_tail(x2, emb, out1):
    R = x2.shape[0]
    off = (R - _S) // _S

    def body(x_ref, e_ref, o1_ref, out_ref):
        out_ref[:, : _D] = x_ref[...]
        out_ref[:, _D:] = e_ref[...]

    return pl.pallas_call(
        body,
        grid=(1,),
        in_specs=[
            pl.BlockSpec((_S, _D), lambda i: (i + off, 0)),
            pl.BlockSpec((_S, _D), lambda i: (i, 0)),
            pl.BlockSpec(memory_space=pl.ANY),
        ],
        out_specs=pl.BlockSpec((_S, 2 * _D), lambda i: (i + off, 0)),
        out_shape=jax.ShapeDtypeStruct((R, 2 * _D), jnp.float32),
        input_output_aliases={2: 0},
    )(x2, emb, out1)


def kernel(x, sem_labels, table, bbox):
    B, N, C = x.shape
    R = B * N
    x2 = x.reshape(R, C)
    labels = sem_labels.reshape(R).astype(jnp.int32)
    emb_tail = _sc_gather_tail(labels, table, R)
    out_tc = _tc_concat_gather_head(x2, labels, table)
    out = _tc_merge_tail(x2, emb_tail, out_tc)
    return out.reshape(B, N, 2 * C)


# speedup vs baseline: 1.0044x; 1.0044x over previous
import functools

import jax
import jax.numpy as jnp
from jax import lax
from jax.experimental import pallas as pl
from jax.experimental.pallas import tpu as pltpu
from jax.experimental.pallas import tpu_sc as plsc

_NUM_CLASSES = 150
_D = 768
_S = 1024
_BM = 1536


def _sc_gather_tail(labels, table, R):
    info = plsc.get_sparse_core_info()
    NC, NS = info.num_cores, info.num_subcores
    NW = NC * NS
    b_per_w = _S // NW
    CH = 32
    n_ch = b_per_w // CH
    S0 = R - _S

    mesh = plsc.VectorSubcoreMesh(core_axis_name="c", subcore_axis_name="s")

    @functools.partial(
        pl.kernel,
        mesh=mesh,
        out_type=jax.ShapeDtypeStruct((_S, _D), jnp.float32),
        scratch_types=[
            pltpu.VMEM((b_per_w,), jnp.int32),
            pltpu.VMEM((CH, _D), jnp.float32),
            pltpu.VMEM((CH, _D), jnp.float32),
            pltpu.SemaphoreType.DMA,
            pltpu.SemaphoreType.DMA,
            pltpu.SemaphoreType.DMA,
            pltpu.SemaphoreType.DMA,
        ],
    )
    def k(lab_hbm, tab_hbm, out_hbm, idx_v, b0, b1, is0, is1, os0, os1):
        bufs = (b0, b1)
        isems = (is0, is1)
        osems = (os0, os1)
        wid = lax.axis_index("s") * NC + lax.axis_index("c")
        base = wid * b_per_w
        pltpu.sync_copy(lab_hbm.at[pl.ds(S0 + base, b_per_w)], idx_v)
        prev_out = [None, None]
        for c in range(n_ch):
            i = c & 1
            r0 = base + c * CH
            if prev_out[i] is not None:
                prev_out[i].wait()
            e_cp = pltpu.make_async_copy(
                tab_hbm.at[idx_v.at[pl.ds(c * CH, CH)]], bufs[i], isems[i])
            e_cp.start()
            e_cp.wait()
            o_cp = pltpu.make_async_copy(
                bufs[i], out_hbm.at[pl.ds(r0, CH), :], osems[i])
            o_cp.start()
            prev_out[i] = o_cp
        prev_out[0].wait()
        if prev_out[1] is not None:
            prev_out[1].wait()

    return k(labels, table)


def _tc_concat_gather_head(x2, labels, table):
    R = x2.shape[0]
    nb = (R - _S) // _BM
    lab3 = labels[: R - _S].reshape(nb, 1, _BM)

    def body(lab_ref, x_ref, tab_ref, out_ref):
        out_ref[:, : _D] = x_ref[...]
        lab = lab_ref[0, 0, :]
        onehot = (lab[:, None] == lax.broadcasted_iota(
            jnp.int32, (_BM, _NUM_CLASSES), 1)).astype(jnp.float32)
        out_ref[:, _D:] = jnp.dot(
            onehot, tab_ref[...], preferred_element_type=jnp.float32)

    return pl.pallas_call(
        body,
        grid=(nb,),
        in_specs=[
            pl.BlockSpec((1, 1, _BM), lambda i: (i, 0, 0)),
            pl.BlockSpec((_BM, _D), lambda i: (i, 0)),
            pl.BlockSpec((_NUM_CLASSES, _D), lambda i: (0, 0)),
        ],
        out_specs=pl.BlockSpec((_BM, 2 * _D), lambda i: (i, 0)),
        out_shape=jax.ShapeDtypeStruct((R, 2 * _D), jnp.float32),
    )(lab3, x2, table)


def _tc_merge_tail(x2, emb, out1):
    R = x2.shape[0]
    off = (R - _S) // _S

    def body(x_ref, e_ref, o1_ref, out_ref):
        out_ref[:, : _D] = x_ref[...]
        out_ref[:, _D:] = e_ref[...]

    return pl.pallas_call(
        body,
        grid=(1,),
        in_specs=[
            pl.BlockSpec((_S, _D), lambda i: (i + off, 0)),
            pl.BlockSpec((_S, _D), lambda i: (i, 0)),
            pl.BlockSpec(memory_space=pl.ANY),
        ],
        out_specs=pl.BlockSpec((_S, 2 * _D), lambda i: (i + off, 0)),
        out_shape=jax.ShapeDtypeStruct((R, 2 * _D), jnp.float32),
        input_output_aliases={2: 0},
    )(x2, emb, out1)


def kernel(x, sem_labels, table, bbox):
    B, N, C = x.shape
    R = B * N
    x2 = x.reshape(R, C)
    labels = sem_labels.reshape(R).astype(jnp.int32)
    emb_tail = _sc_gather_tail(labels, table, R)
    out_tc = _tc_concat_gather_head(x2, labels, table)
    out = _tc_merge_tail(x2, emb_tail, out_tc)
    return out.reshape(B, N, 2 * C)
